# baseline (speedup 1.0000x reference)
import functools

import jax
import jax.numpy as jnp
from jax import lax
from jax.experimental import pallas as pl
from jax.experimental.pallas import tpu as pltpu
from jax.experimental.pallas import tpu_sc as plsc

_L = 16
_BISECT_ITERS = 30


def _row_sparsemax(row_v, cand_v, out_v, n_cols):
    n_chunks = n_cols // _L

    def max_body(i, m):
        return jnp.maximum(m, row_v[pl.ds(i * _L, _L)])

    m = lax.fori_loop(1, n_chunks, max_body, row_v[pl.ds(0, _L)])
    mx = jnp.max(m)
    thr = mx - 1.0

    def compact_body(i, off):
        v = row_v[pl.ds(i * _L, _L)]
        msk = v > thr
        plsc.store_compressed(cand_v.at[pl.ds(off, _L)], v, mask=msk)
        cnt = plsc.all_reduce_population_count(msk)
        return off + jnp.max(cnt)

    off = lax.fori_loop(0, n_chunks, compact_body, jnp.int32(0))
    cand_v[pl.ds(off, _L)] = jnp.full((_L,), thr - 1.0, jnp.float32)
    n_cand_chunks = (off + _L - 1) // _L

    def bisect_body(_, carry):
        lo, hi = carry
        mid = 0.5 * (lo + hi)

        def sum_body(i, acc):
            v = cand_v[pl.ds(i * _L, _L)]
            return acc + jnp.maximum(v - mid, 0.0)

        s = jnp.sum(lax.fori_loop(0, n_cand_chunks, sum_body,
                                  jnp.zeros((_L,), jnp.float32)))
        pred = s >= 1.0
        return jnp.where(pred, mid, lo), jnp.where(pred, hi, mid)

    lo, _ = lax.fori_loop(0, _BISECT_ITERS, bisect_body, (thr, mx))

    def refine_body(i, carry):
        s, c = carry
        v = cand_v[pl.ds(i * _L, _L)]
        msk = v > lo
        s = s + jnp.where(msk, v, 0.0)
        c = c + jnp.max(plsc.all_reduce_population_count(msk))
        return s, c

    s, c = lax.fori_loop(0, n_cand_chunks, refine_body,
                         (jnp.zeros((_L,), jnp.float32), jnp.int32(0)))
    num_v = jnp.full((_L,), jnp.sum(s) - 1.0, jnp.float32)
    den_v = jnp.full((_L,), c.astype(jnp.float32), jnp.float32)
    tau = (num_v / den_v)[0]

    def out_body(i, _):
        v = row_v[pl.ds(i * _L, _L)]
        out_v[pl.ds(i * _L, _L)] = jnp.maximum(v - tau, 0.0)
        return 0

    lax.fori_loop(0, n_chunks, out_body, 0)


def _make_sc_kernel(n_rows, n_cols):
    info = plsc.get_sparse_core_info()
    nw = info.num_cores * info.num_subcores
    rows_per_w = n_rows // nw
    mesh = plsc.VectorSubcoreMesh(core_axis_name="c", subcore_axis_name="s")

    @functools.partial(
        pl.kernel,
        out_type=jax.ShapeDtypeStruct((n_rows, n_cols), jnp.float32),
        mesh=mesh,
        scratch_types=[
            pltpu.VMEM((n_cols,), jnp.float32),
            pltpu.VMEM((n_cols + _L,), jnp.float32),
            pltpu.VMEM((n_cols,), jnp.float32),
        ],
        compiler_params=pltpu.CompilerParams(needs_layout_passes=False),
    )
    def k(x_hbm, out_hbm, row_v, cand_v, out_v):
        wid = lax.axis_index("s") * info.num_cores + lax.axis_index("c")
        for r in range(rows_per_w):
            row = wid * rows_per_w + r
            pltpu.sync_copy(x_hbm.at[row], row_v)
            _row_sparsemax(row_v, cand_v, out_v, n_cols)
            pltpu.sync_copy(out_v, out_hbm.at[row])

    return k


def kernel(x):
    n_rows, n_cols = x.shape
    return _make_sc_kernel(n_rows, n_cols)(x)


# speedup vs baseline: 1.1436x; 1.1436x over previous
import functools

import jax
import jax.numpy as jnp
from jax import lax
from jax.experimental import pallas as pl
from jax.experimental.pallas import tpu as pltpu
from jax.experimental.pallas import tpu_sc as plsc

_L = 16
_U = 4
_BISECT_ITERS = 30


def _row_sparsemax(row_v, cand_v, out_v, n_cols):
    n_chunks = n_cols // _L

    def fused_body(i, carry):
        m, off = carry
        for k in range(_U):
            v = row_v[pl.ds((i * _U + k) * _L, _L)]
            m = jnp.maximum(m, v)
            msk = v > m - 1.0
            plsc.store_compressed(cand_v.at[pl.ds(off, _L)], v, mask=msk)
            off = off + jnp.max(plsc.all_reduce_population_count(msk))
        return m, off

    m0 = jnp.full((_L,), -jnp.inf, jnp.float32)
    m, off = lax.fori_loop(0, n_chunks // _U, fused_body, (m0, jnp.int32(0)))
    mx = jnp.max(m)
    thr = mx - 1.0

    cand_v[pl.ds(off, _L)] = jnp.full((_L,), thr - 1.0, jnp.float32)
    n_sup_chunks = (off + _L - 1) // _L

    def recompact_body(i, off2):
        v = cand_v[pl.ds(i * _L, _L)]
        msk = v > thr
        plsc.store_compressed(cand_v.at[pl.ds(off2, _L)], v, mask=msk)
        return off2 + jnp.max(plsc.all_reduce_population_count(msk))

    off2 = lax.fori_loop(0, n_sup_chunks, recompact_body, jnp.int32(0))
    cand_v[pl.ds(off2, _L)] = jnp.full((_L,), thr - 1.0, jnp.float32)
    n_cand_chunks = (off2 + _L - 1) // _L

    def bisect_body(_, carry):
        lo, hi = carry
        mid = 0.5 * (lo + hi)

        def sum_body(i, acc):
            v = cand_v[pl.ds(i * _L, _L)]
            return acc + jnp.maximum(v - mid, 0.0)

        s = jnp.sum(lax.fori_loop(0, n_cand_chunks, sum_body,
                                  jnp.zeros((_L,), jnp.float32)))
        pred = s >= 1.0
        return jnp.where(pred, mid, lo), jnp.where(pred, hi, mid)

    lo, _ = lax.fori_loop(0, _BISECT_ITERS, bisect_body, (thr, mx))

    def refine_body(i, carry):
        s, c = carry
        v = cand_v[pl.ds(i * _L, _L)]
        msk = v > lo
        s = s + jnp.where(msk, v, 0.0)
        c = c + jnp.max(plsc.all_reduce_population_count(msk))
        return s, c

    s, c = lax.fori_loop(0, n_cand_chunks, refine_body,
                         (jnp.zeros((_L,), jnp.float32), jnp.int32(0)))
    num_v = jnp.full((_L,), jnp.sum(s) - 1.0, jnp.float32)
    den_v = jnp.full((_L,), c.astype(jnp.float32), jnp.float32)
    tau = (num_v / den_v)[0]

    def out_body(i, _):
        for k in range(_U):
            v = row_v[pl.ds((i * _U + k) * _L, _L)]
            out_v[pl.ds((i * _U + k) * _L, _L)] = jnp.maximum(v - tau, 0.0)
        return 0

    lax.fori_loop(0, n_chunks // _U, out_body, 0)


def _make_sc_kernel(n_rows, n_cols):
    info = plsc.get_sparse_core_info()
    nw = info.num_cores * info.num_subcores
    rows_per_w = n_rows // nw
    mesh = plsc.VectorSubcoreMesh(core_axis_name="c", subcore_axis_name="s")

    @functools.partial(
        pl.kernel,
        out_type=jax.ShapeDtypeStruct((n_rows, n_cols), jnp.float32),
        mesh=mesh,
        scratch_types=[
            pltpu.VMEM((n_cols,), jnp.float32),
            pltpu.VMEM((n_cols + _L,), jnp.float32),
            pltpu.VMEM((n_cols,), jnp.float32),
        ],
        compiler_params=pltpu.CompilerParams(needs_layout_passes=False),
    )
    def k(x_hbm, out_hbm, row_v, cand_v, out_v):
        wid = lax.axis_index("s") * info.num_cores + lax.axis_index("c")
        for r in range(rows_per_w):
            row = wid * rows_per_w + r
            pltpu.sync_copy(x_hbm.at[row], row_v)
            _row_sparsemax(row_v, cand_v, out_v, n_cols)
            pltpu.sync_copy(out_v, out_hbm.at[row])

    return k


def kernel(x):
    n_rows, n_cols = x.shape
    return _make_sc_kernel(n_rows, n_cols)(x)


# speedup vs baseline: 1.2295x; 1.0751x over previous
import functools

import jax
import jax.numpy as jnp
from jax import lax
from jax.experimental import pallas as pl
from jax.experimental.pallas import tpu as pltpu
from jax.experimental.pallas import tpu_sc as plsc

_L = 16
_U = 8
_BISECT_ITERS = 30


def _row_sparsemax(row_v, cand_v, out_v, n_cols):
    n_chunks = n_cols // _L

    def fused_body(i, carry):
        m, off = carry
        for k in range(_U):
            v = row_v[pl.ds((i * _U + k) * _L, _L)]
            m = jnp.maximum(m, v)
            msk = v > m - 1.0
            plsc.store_compressed(cand_v.at[pl.ds(off, _L)], v, mask=msk)
            off = off + plsc.all_reduce_population_count(msk)[0]
        return m, off

    m0 = jnp.full((_L,), -jnp.inf, jnp.float32)
    m, off = lax.fori_loop(0, n_chunks // _U, fused_body, (m0, jnp.int32(0)))
    mx = jnp.max(m)
    thr = mx - 1.0

    cand_v[pl.ds(off, _L)] = jnp.full((_L,), thr - 1.0, jnp.float32)
    n_sup_chunks = (off + _L - 1) // _L

    def recompact_body(i, off2):
        v = cand_v[pl.ds(i * _L, _L)]
        msk = v > thr
        plsc.store_compressed(cand_v.at[pl.ds(off2, _L)], v, mask=msk)
        return off2 + plsc.all_reduce_population_count(msk)[0]

    off2 = lax.fori_loop(0, n_sup_chunks, recompact_body, jnp.int32(0))
    cand_v[pl.ds(off2, _L)] = jnp.full((_L,), thr - 1.0, jnp.float32)
    n_cand_chunks = (off2 + _L - 1) // _L

    def bisect_body(_, carry):
        lo, hi = carry
        mid = 0.5 * (lo + hi)

        def sum_body(i, acc):
            v = cand_v[pl.ds(i * _L, _L)]
            return acc + jnp.maximum(v - mid, 0.0)

        s = jnp.sum(lax.fori_loop(0, n_cand_chunks, sum_body,
                                  jnp.zeros((_L,), jnp.float32)))
        pred = s >= 1.0
        return jnp.where(pred, mid, lo), jnp.where(pred, hi, mid)

    lo, _ = lax.fori_loop(0, _BISECT_ITERS, bisect_body, (thr, mx))

    def refine_body(i, carry):
        s, c = carry
        v = cand_v[pl.ds(i * _L, _L)]
        msk = v > lo
        s = s + jnp.where(msk, v, 0.0)
        c = c + plsc.all_reduce_population_count(msk)[0]
        return s, c

    s, c = lax.fori_loop(0, n_cand_chunks, refine_body,
                         (jnp.zeros((_L,), jnp.float32), jnp.int32(0)))
    num_v = jnp.full((_L,), jnp.sum(s) - 1.0, jnp.float32)
    den_v = jnp.full((_L,), c.astype(jnp.float32), jnp.float32)
    tau = (num_v / den_v)[0]

    def out_body(i, _):
        for k in range(_U):
            v = row_v[pl.ds((i * _U + k) * _L, _L)]
            out_v[pl.ds((i * _U + k) * _L, _L)] = jnp.maximum(v - tau, 0.0)
        return 0

    lax.fori_loop(0, n_chunks // _U, out_body, 0)


def _make_sc_kernel(n_rows, n_cols):
    info = plsc.get_sparse_core_info()
    nw = info.num_cores * info.num_subcores
    rows_per_w = n_rows // nw
    mesh = plsc.VectorSubcoreMesh(core_axis_name="c", subcore_axis_name="s")

    @functools.partial(
        pl.kernel,
        out_type=jax.ShapeDtypeStruct((n_rows, n_cols), jnp.float32),
        mesh=mesh,
        scratch_types=[
            pltpu.VMEM((n_cols,), jnp.float32),
            pltpu.VMEM((n_cols + _L,), jnp.float32),
            pltpu.VMEM((n_cols,), jnp.float32),
        ],
        compiler_params=pltpu.CompilerParams(needs_layout_passes=False),
    )
    def k(x_hbm, out_hbm, row_v, cand_v, out_v):
        wid = lax.axis_index("s") * info.num_cores + lax.axis_index("c")
        for r in range(rows_per_w):
            row = wid * rows_per_w + r
            pltpu.sync_copy(x_hbm.at[row], row_v)
            _row_sparsemax(row_v, cand_v, out_v, n_cols)
            pltpu.sync_copy(out_v, out_hbm.at[row])

    return k


def kernel(x):
    n_rows, n_cols = x.shape
    return _make_sc_kernel(n_rows, n_cols)(x)


# speedup vs baseline: 1.2579x; 1.0231x over previous
import functools

import jax
import jax.numpy as jnp
from jax import lax
from jax.experimental import pallas as pl
from jax.experimental.pallas import tpu as pltpu
from jax.experimental.pallas import tpu_sc as plsc

_L = 16
_U = 8
_BISECT_ITERS = 30


def _row_sparsemax(row_v, cand_v, out_v, n_cols):
    n_chunks = n_cols // _L

    def fused_body(i, carry):
        m, off = carry
        for k in range(_U):
            v = row_v[pl.ds((i * _U + k) * _L, _L)]
            m = jnp.maximum(m, v)
            msk = v > m - 1.0
            plsc.store_compressed(cand_v.at[pl.ds(off, _L)], v, mask=msk)
            off = off + plsc.all_reduce_population_count(msk)[0]
        return m, off

    m0 = jnp.full((_L,), -jnp.inf, jnp.float32)
    m, off = lax.fori_loop(0, n_chunks // _U, fused_body, (m0, jnp.int32(0)))
    mx = jnp.max(m)
    thr = mx - 1.0

    cand_v[pl.ds(off, _L)] = jnp.full((_L,), thr - 1.0, jnp.float32)
    n_sup_chunks = (off + _L - 1) // _L

    def recompact_body(i, off2):
        v = cand_v[pl.ds(i * _L, _L)]
        msk = v > thr
        plsc.store_compressed(cand_v.at[pl.ds(off2, _L)], v, mask=msk)
        return off2 + plsc.all_reduce_population_count(msk)[0]

    off2 = lax.fori_loop(0, n_sup_chunks, recompact_body, jnp.int32(0))
    for k in range(4):
        cand_v[pl.ds(off2 + k * _L, _L)] = jnp.full((_L,), thr - 1.0,
                                                    jnp.float32)
    n_cand_chunks = (off2 + _L - 1) // _L

    c0 = cand_v[pl.ds(0, _L)]
    c1 = cand_v[pl.ds(1 * _L, _L)]
    c2 = cand_v[pl.ds(2 * _L, _L)]
    c3 = cand_v[pl.ds(3 * _L, _L)]

    def bisect_body(_, carry):
        lo_v, hi_v = carry
        mid_v = 0.5 * (lo_v + hi_v)
        s_vec = (jnp.maximum(c0 - mid_v, 0.0) + jnp.maximum(c1 - mid_v, 0.0) +
                 jnp.maximum(c2 - mid_v, 0.0) + jnp.maximum(c3 - mid_v, 0.0))

        def sum_body(i, acc):
            v = cand_v[pl.ds(i * _L, _L)]
            return acc + jnp.maximum(v - mid_v, 0.0)

        s = jnp.sum(lax.fori_loop(4, n_cand_chunks, sum_body, s_vec))
        pred = s >= 1.0
        return jnp.where(pred, mid_v, lo_v), jnp.where(pred, hi_v, mid_v)

    thr_v = jnp.full((_L,), thr, jnp.float32)
    mx_v = jnp.full((_L,), mx, jnp.float32)
    lo_v, _ = lax.fori_loop(0, _BISECT_ITERS, bisect_body, (thr_v, mx_v))

    s_vec = jnp.zeros((_L,), jnp.float32)
    c_cnt = jnp.int32(0)
    for cv in (c0, c1, c2, c3):
        msk = cv > lo_v
        s_vec = s_vec + jnp.where(msk, cv, 0.0)
        c_cnt = c_cnt + plsc.all_reduce_population_count(msk)[0]

    def refine_body(i, carry):
        s, c = carry
        v = cand_v[pl.ds(i * _L, _L)]
        msk = v > lo_v
        s = s + jnp.where(msk, v, 0.0)
        c = c + plsc.all_reduce_population_count(msk)[0]
        return s, c

    s_vec, c_cnt = lax.fori_loop(4, n_cand_chunks, refine_body,
                                 (s_vec, c_cnt))
    num_v = jnp.full((_L,), jnp.sum(s_vec) - 1.0, jnp.float32)
    den_v = jnp.full((_L,), c_cnt.astype(jnp.float32), jnp.float32)
    tau_v = num_v / den_v

    def out_body(i, _):
        for k in range(_U):
            v = row_v[pl.ds((i * _U + k) * _L, _L)]
            out_v[pl.ds((i * _U + k) * _L, _L)] = jnp.maximum(v - tau_v, 0.0)
        return 0

    lax.fori_loop(0, n_chunks // _U, out_body, 0)


def _make_sc_kernel(n_rows, n_cols):
    info = plsc.get_sparse_core_info()
    nw = info.num_cores * info.num_subcores
    rows_per_w = n_rows // nw
    mesh = plsc.VectorSubcoreMesh(core_axis_name="c", subcore_axis_name="s")

    @functools.partial(
        pl.kernel,
        out_type=jax.ShapeDtypeStruct((n_rows, n_cols), jnp.float32),
        mesh=mesh,
        scratch_types=[
            pltpu.VMEM((n_cols,), jnp.float32),
            pltpu.VMEM((n_cols + 4 * _L,), jnp.float32),
            pltpu.VMEM((n_cols,), jnp.float32),
        ],
        compiler_params=pltpu.CompilerParams(needs_layout_passes=False),
    )
    def k(x_hbm, out_hbm, row_v, cand_v, out_v):
        wid = lax.axis_index("s") * info.num_cores + lax.axis_index("c")
        for r in range(rows_per_w):
            row = wid * rows_per_w + r
            pltpu.sync_copy(x_hbm.at[row], row_v)
            _row_sparsemax(row_v, cand_v, out_v, n_cols)
            pltpu.sync_copy(out_v, out_hbm.at[row])

    return k


def kernel(x):
    n_rows, n_cols = x.shape
    return _make_sc_kernel(n_rows, n_cols)(x)
